# baseline (speedup 1.0000x reference)
import functools

import jax
import jax.numpy as jnp
from jax import lax
from jax.experimental import pallas as pl
from jax.experimental.pallas import tpu as pltpu
from jax.experimental.pallas import tpu_sc as plsc

_BLOCK_M = 128
_COMMITMENT_COST = 0.25
_ENTROPY_LOSS_RATIO = 0.1
_ENTROPY_TEMPERATURE = 0.01


def _vq_body(nsteps, total_rows, e_dim, x_ref, cbt_ref, idx_ref, e_ref, q_ref,
             ent_ref, loss_ref, b2_ref, avgp_ref, sent_ref, mind_ref):
    i = pl.program_id(0)

    @pl.when(i == 0)
    def _init():
        cbt = cbt_ref[...]
        b2_ref[...] = jnp.sum(cbt * cbt, axis=0, keepdims=True)
        avgp_ref[...] = jnp.zeros_like(avgp_ref)
        sent_ref[...] = jnp.zeros_like(sent_ref)
        mind_ref[...] = jnp.zeros_like(mind_ref)

    xb = x_ref[...]
    a2 = jnp.sum(xb * xb, axis=1, keepdims=True)
    ab = jnp.dot(xb, cbt_ref[...], preferred_element_type=jnp.float32)
    dist = (a2 - 2.0 * ab) + b2_ref[...]

    mind = jnp.min(dist, axis=1, keepdims=True)
    kidx = lax.broadcasted_iota(jnp.int32, dist.shape, 1)
    idx = jnp.min(jnp.where(dist == mind, kidx, dist.shape[1]),
                  axis=1, keepdims=True)
    idx_ref[...] = idx

    u = mind - dist
    p = jnp.exp2(u * (1.4426950408889634 / _ENTROPY_TEMPERATURE))
    z = jnp.sum(p, axis=1, keepdims=True)
    rz = 1.0 / z
    avgp_ref[...] += jnp.sum(p * rz, axis=0, keepdims=True)
    row_plogp = (jnp.sum(p * u, axis=1, keepdims=True)
                 * (rz * (1.0 / _ENTROPY_TEMPERATURE)) - jnp.log(z))
    sent_ref[...] += jnp.sum(row_plogp, keepdims=True)
    mind_ref[...] += jnp.sum(mind, keepdims=True)

    @pl.when(i == nsteps - 1)
    def _finalize():
        n = float(total_rows)
        avgp = avgp_ref[...] * (1.0 / n)
        avg_ent = -jnp.sum(avgp * jnp.log(avgp + 1e-5), keepdims=True)
        samp_ent = -(sent_ref[...] * (1.0 / n))
        ent = (samp_ent - avg_ent) * _ENTROPY_LOSS_RATIO
        mse = mind_ref[...] * (1.0 / (n * e_dim))
        e = mse * _COMMITMENT_COST
        q = mse
        e_ref[...] = e
        q_ref[...] = q
        ent_ref[...] = ent
        loss_ref[...] = (e + q) + ent


def _vq_stats(xf, cbt):
    m, e_dim = xf.shape
    k = cbt.shape[1]
    nsteps = m // _BLOCK_M
    body = functools.partial(_vq_body, nsteps, m, e_dim)
    scalar = jax.ShapeDtypeStruct((1, 1), jnp.float32)
    scalar_spec = pl.BlockSpec((1, 1), lambda i: (0, 0))
    return pl.pallas_call(
        body,
        grid=(nsteps,),
        in_specs=[
            pl.BlockSpec((_BLOCK_M, e_dim), lambda i: (i, 0)),
            pl.BlockSpec((e_dim, k), lambda i: (0, 0)),
        ],
        out_specs=[
            pl.BlockSpec((_BLOCK_M, 1), lambda i: (i, 0)),
            scalar_spec, scalar_spec, scalar_spec, scalar_spec,
        ],
        out_shape=[
            jax.ShapeDtypeStruct((m, 1), jnp.int32),
            scalar, scalar, scalar, scalar,
        ],
        scratch_shapes=[
            pltpu.VMEM((1, k), jnp.float32),
            pltpu.VMEM((1, k), jnp.float32),
            pltpu.VMEM((1, 1), jnp.float32),
            pltpu.VMEM((1, 1), jnp.float32),
        ],
    )(xf, cbt)


def _sc_gather(table, idx_flat):
    b = idx_flat.shape[0]
    d = table.shape[1]
    nc, ns = 2, 16
    bpw = b // (nc * ns)
    mesh = plsc.VectorSubcoreMesh(core_axis_name="c", subcore_axis_name="s")

    @functools.partial(
        pl.kernel, mesh=mesh,
        out_type=jax.ShapeDtypeStruct((b, d), jnp.float32),
        scratch_types=[
            pltpu.VMEM((bpw,), jnp.int32),
            pltpu.VMEM((bpw, d), jnp.float32),
            pltpu.SemaphoreType.DMA,
        ],
    )
    def k(table_hbm, idx_hbm, out_hbm, idx_v, rows_v, sem):
        wid = lax.axis_index("s") * nc + lax.axis_index("c")
        base = wid * bpw
        pltpu.sync_copy(idx_hbm.at[pl.ds(base, bpw)], idx_v)
        pltpu.async_copy(table_hbm.at[idx_v], rows_v, sem).wait()
        pltpu.sync_copy(rows_v, out_hbm.at[pl.ds(base, bpw)])

    return k(table, idx_flat)


def kernel(x, codebook):
    bsz, tok, e_dim = x.shape
    xf = x.reshape(-1, e_dim)
    cbt = codebook.T
    idx2, e2, q2, ent2, loss2 = _vq_stats(xf, cbt)
    idx_flat = idx2.reshape(-1)
    quant = _sc_gather(codebook, idx_flat)
    return (
        quant.reshape(x.shape),
        loss2[0, 0],
        e2[0, 0],
        q2[0, 0],
        ent2[0, 0],
        idx_flat.reshape(bsz, tok),
    )


# speedup vs baseline: 1.1206x; 1.1206x over previous
import functools

import jax
import jax.numpy as jnp
from jax import lax
from jax.experimental import pallas as pl
from jax.experimental.pallas import tpu as pltpu
from jax.experimental.pallas import tpu_sc as plsc

_BLOCK_M = 256
_COMMITMENT_COST = 0.25
_ENTROPY_LOSS_RATIO = 0.1
_ENTROPY_TEMPERATURE = 0.01


def _vq_body(nsteps, total_rows, e_dim, x_ref, cbt_ref, idx_ref, e_ref, q_ref,
             ent_ref, loss_ref, b2_ref, avgp_ref, sent_ref, mind_ref):
    i = pl.program_id(0)

    @pl.when(i == 0)
    def _init():
        cbt = cbt_ref[...]
        b2_ref[...] = jnp.sum(cbt * cbt, axis=0, keepdims=True)
        avgp_ref[...] = jnp.zeros_like(avgp_ref)
        sent_ref[...] = jnp.zeros_like(sent_ref)
        mind_ref[...] = jnp.zeros_like(mind_ref)

    xb = x_ref[...]
    a2 = jnp.sum(xb * xb, axis=1, keepdims=True)
    ab = jnp.dot(xb, cbt_ref[...], preferred_element_type=jnp.float32)
    dist = (a2 - 2.0 * ab) + b2_ref[...]

    mind = jnp.min(dist, axis=1, keepdims=True)
    kidx = lax.broadcasted_iota(jnp.int32, dist.shape, 1)
    idx = jnp.min(jnp.where(dist == mind, kidx, dist.shape[1]),
                  axis=1, keepdims=True)
    idx_ref[...] = idx

    u = mind - dist
    p = jnp.exp2(u * (1.4426950408889634 / _ENTROPY_TEMPERATURE))
    z = jnp.sum(p, axis=1, keepdims=True)
    rz = 1.0 / z
    avgp_ref[...] += jnp.sum(p * rz, axis=0, keepdims=True)
    row_plogp = (jnp.sum(p * u, axis=1, keepdims=True)
                 * (rz * (1.0 / _ENTROPY_TEMPERATURE)) - jnp.log(z))
    sent_ref[...] += jnp.sum(row_plogp, keepdims=True)
    mind_ref[...] += jnp.sum(mind, keepdims=True)

    @pl.when(i == nsteps - 1)
    def _finalize():
        n = float(total_rows)
        avgp = avgp_ref[...] * (1.0 / n)
        avg_ent = -jnp.sum(avgp * jnp.log(avgp + 1e-5), keepdims=True)
        samp_ent = -(sent_ref[...] * (1.0 / n))
        ent = (samp_ent - avg_ent) * _ENTROPY_LOSS_RATIO
        mse = mind_ref[...] * (1.0 / (n * e_dim))
        e = mse * _COMMITMENT_COST
        q = mse
        e_ref[...] = e
        q_ref[...] = q
        ent_ref[...] = ent
        loss_ref[...] = (e + q) + ent


def _vq_stats(xf, cbt):
    m, e_dim = xf.shape
    k = cbt.shape[1]
    nsteps = m // _BLOCK_M
    body = functools.partial(_vq_body, nsteps, m, e_dim)
    scalar = jax.ShapeDtypeStruct((1, 1), jnp.float32)
    scalar_spec = pl.BlockSpec((1, 1), lambda i: (0, 0))
    return pl.pallas_call(
        body,
        grid=(nsteps,),
        in_specs=[
            pl.BlockSpec((_BLOCK_M, e_dim), lambda i: (i, 0)),
            pl.BlockSpec((e_dim, k), lambda i: (0, 0)),
        ],
        out_specs=[
            pl.BlockSpec((_BLOCK_M, 1), lambda i: (i, 0)),
            scalar_spec, scalar_spec, scalar_spec, scalar_spec,
        ],
        out_shape=[
            jax.ShapeDtypeStruct((m, 1), jnp.int32),
            scalar, scalar, scalar, scalar,
        ],
        scratch_shapes=[
            pltpu.VMEM((1, k), jnp.float32),
            pltpu.VMEM((1, k), jnp.float32),
            pltpu.VMEM((1, 1), jnp.float32),
            pltpu.VMEM((1, 1), jnp.float32),
        ],
    )(xf, cbt)


def _sc_gather(table, idx_flat):
    b = idx_flat.shape[0]
    d = table.shape[1]
    nc, ns = 2, 16
    bpw = b // (nc * ns)
    mesh = plsc.VectorSubcoreMesh(core_axis_name="c", subcore_axis_name="s")

    @functools.partial(
        pl.kernel, mesh=mesh,
        out_type=jax.ShapeDtypeStruct((b, d), jnp.float32),
        scratch_types=[
            pltpu.VMEM((bpw,), jnp.int32),
            pltpu.VMEM((bpw, d), jnp.float32),
            pltpu.SemaphoreType.DMA,
        ],
    )
    def k(table_hbm, idx_hbm, out_hbm, idx_v, rows_v, sem):
        wid = lax.axis_index("s") * nc + lax.axis_index("c")
        base = wid * bpw
        pltpu.sync_copy(idx_hbm.at[pl.ds(base, bpw)], idx_v)
        pltpu.async_copy(table_hbm.at[idx_v], rows_v, sem).wait()
        pltpu.sync_copy(rows_v, out_hbm.at[pl.ds(base, bpw)])

    return k(table, idx_flat)


def kernel(x, codebook):
    bsz, tok, e_dim = x.shape
    xf = x.reshape(-1, e_dim)
    cbt = codebook.T
    idx2, e2, q2, ent2, loss2 = _vq_stats(xf, cbt)
    idx_flat = idx2.reshape(-1)
    quant = _sc_gather(codebook, idx_flat)
    return (
        quant.reshape(x.shape),
        loss2[0, 0],
        e2[0, 0],
        q2[0, 0],
        ent2[0, 0],
        idx_flat.reshape(bsz, tok),
    )
